# baseline (device time: 61501 ns/iter reference)
import jax
import jax.numpy as jnp
from jax import lax
from jax.experimental import pallas as pl
from jax.experimental.pallas import tpu as pltpu

N_DEV = 4
B, SQ, D = 4, 256, 1024
HQ_LOCAL = 8
DH = 128
SKV = 1024
HALF = SQ // 2
SCALE = 0.08838834764831843
BF = jnp.bfloat16


def kernel(x, Wq, Wo, K_ext, V_ext):
    x2 = x.reshape(B * SQ, D)

    def body(x_ref, wq_ref, wo_ref, k_any, v_any, out_ref,
             x_bf, wq_bf, wo_bf, k_heads, v_heads, k_bf, v_aug,
             q_bf, attn_bf, rs_out, rs_buf, ag_bf,
             kv_sems, rs_send, rs_recv, ag_sems):
        my = lax.axis_index("i")
        left = lax.rem(my + N_DEV - 1, N_DEV)
        right = lax.rem(my + 1, N_DEV)
        opp = lax.rem(my + 2, N_DEV)

        barrier = pltpu.get_barrier_semaphore()
        for nbr in (left, right, opp):
            pl.semaphore_signal(barrier, inc=1, device_id=(nbr,),
                                device_id_type=pl.DeviceIdType.MESH)
        pl.semaphore_wait(barrier, 3)

        for b in range(B):
            for kvl in range(2):
                g = 2 * my + kvl
                pltpu.make_async_copy(
                    k_any.at[b, :, g, :], k_heads.at[b * 2 + kvl],
                    kv_sems.at[b, kvl]).start()
                pltpu.make_async_copy(
                    v_any.at[b, :, g, :], v_heads.at[b * 2 + kvl],
                    kv_sems.at[b, 2 + kvl]).start()

        x_bf[...] = x_ref[...].astype(BF)
        wq_bf[...] = wq_ref[...].astype(BF)
        wo_bf[...] = wo_ref[...].astype(BF)

        colz = lax.broadcasted_iota(jnp.int32, (SKV, DH), 1)
        ones_col = jnp.where(colz == 0, 1.0, 0.0).astype(BF)
        v_aug[0, :, DH:] = ones_col
        v_aug[1, :, DH:] = ones_col

        def wait_kv(b):
            for kvl in range(2):
                g = 2 * my + kvl
                pltpu.make_async_copy(
                    k_any.at[b, :, g, :], k_heads.at[b * 2 + kvl],
                    kv_sems.at[b, kvl]).wait()
                pltpu.make_async_copy(
                    v_any.at[b, :, g, :], v_heads.at[b * 2 + kvl],
                    kv_sems.at[b, 2 + kvl]).wait()
                k_bf[kvl] = k_heads[b * 2 + kvl].astype(BF)
                v_aug[kvl, :, :DH] = v_heads[b * 2 + kvl].astype(BF)

        def compute_rows(row0, nrows):
            rows = pl.ds(row0, nrows)
            qs = pl.ds(0, nrows)
            q_bf[qs, :] = (jnp.dot(x_bf[rows, :], wq_bf[...],
                                   preferred_element_type=jnp.float32)
                           * SCALE).astype(BF)
            for kvl in range(2):
                cols = pl.ds(kvl * 4 * DH, 4 * DH)
                q3 = q_bf[qs, cols].reshape(nrows * 4, DH)
                s3 = lax.dot_general(
                    q3, k_bf[kvl], (((1,), (1,)), ((), ())),
                    preferred_element_type=jnp.float32)
                p3 = jnp.exp(s3).astype(BF)
                o_aug = jnp.dot(p3, v_aug[kvl],
                                preferred_element_type=jnp.float32)
                o = o_aug[:, :DH] / o_aug[:, DH:DH + 1]
                attn_bf[qs, cols] = o.astype(BF).reshape(nrows, 4 * DH)
            out_ref[rows, :] = jnp.dot(attn_bf[qs, :], wo_bf[...],
                                       preferred_element_type=jnp.float32)

        wait_kv(my)
        compute_rows(my * SQ, SQ)
        rs_out[0] = out_ref[pl.ds(my * SQ, SQ), :].astype(BF)
        prev = pltpu.make_async_remote_copy(
            src_ref=rs_out.at[0], dst_ref=rs_buf.at[0],
            send_sem=rs_send.at[0], recv_sem=rs_recv.at[0],
            device_id=(right,), device_id_type=pl.DeviceIdType.MESH,
        )
        prev.start()
        for idx in (1, 2):
            b = lax.rem(my - idx + N_DEV, N_DEV)
            wait_kv(b)
            compute_rows(b * SQ, SQ)
            prev.wait()
            rows = pl.ds(b * SQ, SQ)
            rs_out[idx] = (out_ref[rows, :]
                           + rs_buf[idx - 1].astype(jnp.float32)).astype(BF)
            prev = pltpu.make_async_remote_copy(
                src_ref=rs_out.at[idx], dst_ref=rs_buf.at[idx],
                send_sem=rs_send.at[idx], recv_sem=rs_recv.at[idx],
                device_id=(right,), device_id_type=pl.DeviceIdType.MESH,
            )
            prev.start()

        b3 = lax.rem(my + 1, N_DEV)
        wait_kv(b3)
        prev.wait()
        ag_rdmas = []
        for h in range(2):
            compute_rows(b3 * SQ + h * HALF, HALF)
            r128 = pl.ds(b3 * SQ + h * HALF, HALF)
            acc = (out_ref[r128, :]
                   + rs_buf[2, pl.ds(h * HALF, HALF), :].astype(jnp.float32))
            out_ref[r128, :] = acc
            ag_bf[0, pl.ds(h * HALF, HALF), :] = acc.astype(BF)
            for j, (tgt, slot) in enumerate(
                    ((right, 1), (left, 2), (opp, 3))):
                d = pltpu.make_async_remote_copy(
                    src_ref=ag_bf.at[0, pl.ds(h * HALF, HALF)],
                    dst_ref=ag_bf.at[slot, pl.ds(h * HALF, HALF)],
                    send_sem=ag_sems.at[6 * h + 2 * j],
                    recv_sem=ag_sems.at[6 * h + 2 * j + 1],
                    device_id=(tgt,), device_id_type=pl.DeviceIdType.MESH,
                )
                d.start()
                ag_rdmas.append(d)
        for d in ag_rdmas:
            d.wait()
        for slot, c in ((1, my), (2, opp), (3, lax.rem(my + 3, N_DEV))):
            out_ref[pl.ds(c * SQ, SQ), :] = ag_bf[slot].astype(jnp.float32)

    out = pl.pallas_call(
        body,
        out_shape=jax.ShapeDtypeStruct((B * SQ, D), jnp.float32),
        in_specs=[
            pl.BlockSpec(memory_space=pltpu.VMEM),
            pl.BlockSpec(memory_space=pltpu.VMEM),
            pl.BlockSpec(memory_space=pltpu.VMEM),
            pl.BlockSpec(memory_space=pl.ANY),
            pl.BlockSpec(memory_space=pl.ANY),
        ],
        out_specs=pl.BlockSpec(memory_space=pltpu.VMEM),
        scratch_shapes=[
            pltpu.VMEM((B * SQ, D), BF),
            pltpu.VMEM((D, D), BF),
            pltpu.VMEM((D, D), BF),
            pltpu.VMEM((B * 2, SKV, DH), jnp.float32),
            pltpu.VMEM((B * 2, SKV, DH), jnp.float32),
            pltpu.VMEM((2, SKV, DH), BF),
            pltpu.VMEM((2, SKV, 2 * DH), BF),
            pltpu.VMEM((SQ, D), BF),
            pltpu.VMEM((SQ, D), BF),
            pltpu.VMEM((N_DEV - 1, SQ, D), BF),
            pltpu.VMEM((N_DEV - 1, SQ, D), BF),
            pltpu.VMEM((N_DEV, SQ, D), BF),
            pltpu.SemaphoreType.DMA((B, 4)),
            pltpu.SemaphoreType.DMA((N_DEV - 1,)),
            pltpu.SemaphoreType.DMA((N_DEV - 1,)),
            pltpu.SemaphoreType.DMA((12,)),
        ],
        compiler_params=pltpu.CompilerParams(
            collective_id=0, vmem_limit_bytes=100 * 1024 * 1024),
    )(x2, Wq, Wo, K_ext, V_ext)
    return out.reshape(B, SQ, D)


# device time: 60873 ns/iter; 1.0103x vs baseline; 1.0103x over previous
import jax
import jax.numpy as jnp
from jax import lax
from jax.experimental import pallas as pl
from jax.experimental.pallas import tpu as pltpu

N_DEV = 4
B, SQ, D = 4, 256, 1024
HQ_LOCAL = 8
DH = 128
SKV = 1024
SCALE = 0.08838834764831843
BF = jnp.bfloat16


def kernel(x, Wq, Wo, K_ext, V_ext):
    x2 = x.reshape(B * SQ, D)

    def body(x_ref, wq_ref, wo_ref, k_any, v_any, out_ref,
             x_bf, wq_bf, wo_bf, k_heads, v_heads, k_bf, v_aug,
             q_bf, attn_bf, rs_out, rs_buf, ag_bf,
             kv_sems, rs_send, rs_recv, ag_sems):
        my = lax.axis_index("i")
        left = lax.rem(my + N_DEV - 1, N_DEV)
        right = lax.rem(my + 1, N_DEV)

        barrier = pltpu.get_barrier_semaphore()
        for nbr in (left, right):
            pl.semaphore_signal(barrier, inc=1, device_id=(nbr,),
                                device_id_type=pl.DeviceIdType.MESH)
        pl.semaphore_wait(barrier, 2)

        for b in range(B):
            for kvl in range(2):
                g = 2 * my + kvl
                pltpu.make_async_copy(
                    k_any.at[b, :, g, :], k_heads.at[b * 2 + kvl],
                    kv_sems.at[b, kvl]).start()
                pltpu.make_async_copy(
                    v_any.at[b, :, g, :], v_heads.at[b * 2 + kvl],
                    kv_sems.at[b, 2 + kvl]).start()

        x_bf[...] = x_ref[...].astype(BF)
        wq_bf[...] = wq_ref[...].astype(BF)
        wo_bf[...] = wo_ref[...].astype(BF)

        colz = lax.broadcasted_iota(jnp.int32, (SKV, DH), 1)
        ones_col = jnp.where(colz == 0, 1.0, 0.0).astype(BF)
        v_aug[0, :, DH:] = ones_col
        v_aug[1, :, DH:] = ones_col

        def compute_chunk(b):
            rows = pl.ds(b * SQ, SQ)
            q_bf[...] = (jnp.dot(x_bf[rows, :], wq_bf[...],
                                 preferred_element_type=jnp.float32)
                         * SCALE).astype(BF)

            for kvl in range(2):
                g = 2 * my + kvl
                pltpu.make_async_copy(
                    k_any.at[b, :, g, :], k_heads.at[b * 2 + kvl],
                    kv_sems.at[b, kvl]).wait()
                pltpu.make_async_copy(
                    v_any.at[b, :, g, :], v_heads.at[b * 2 + kvl],
                    kv_sems.at[b, 2 + kvl]).wait()
                k_bf[kvl] = k_heads[b * 2 + kvl].astype(BF)
                v_aug[kvl, :, :DH] = v_heads[b * 2 + kvl].astype(BF)

            for kvl in range(2):
                cols = pl.ds(kvl * 4 * DH, 4 * DH)
                q3 = q_bf[:, cols].reshape(SQ * 4, DH)
                s3 = lax.dot_general(
                    q3, k_bf[kvl], (((1,), (1,)), ((), ())),
                    preferred_element_type=jnp.float32)
                p3 = jnp.exp(s3).astype(BF)
                o_aug = jnp.dot(p3, v_aug[kvl],
                                preferred_element_type=jnp.float32)
                o = o_aug[:, :DH] / o_aug[:, DH:DH + 1]
                attn_bf[:, cols] = o.astype(BF).reshape(SQ, 4 * DH)

            out_ref[rows, :] = jnp.dot(attn_bf[...], wo_bf[...],
                                       preferred_element_type=jnp.float32)

        compute_chunk(my)
        rs_out[0] = out_ref[pl.ds(my * SQ, SQ), :].astype(BF)
        prev = pltpu.make_async_remote_copy(
            src_ref=rs_out.at[0],
            dst_ref=rs_buf.at[0],
            send_sem=rs_send.at[0], recv_sem=rs_recv.at[0],
            device_id=(right,), device_id_type=pl.DeviceIdType.MESH,
        )
        prev.start()
        for idx in range(1, N_DEV):
            b = lax.rem(my - idx + N_DEV, N_DEV)
            compute_chunk(b)
            prev.wait()
            rows = pl.ds(b * SQ, SQ)
            acc = out_ref[rows, :] + rs_buf[idx - 1].astype(jnp.float32)
            if idx < N_DEV - 1:
                rs_out[idx] = acc.astype(BF)
                prev = pltpu.make_async_remote_copy(
                    src_ref=rs_out.at[idx],
                    dst_ref=rs_buf.at[idx],
                    send_sem=rs_send.at[idx], recv_sem=rs_recv.at[idx],
                    device_id=(right,), device_id_type=pl.DeviceIdType.MESH,
                )
                prev.start()
            else:
                out_ref[rows, :] = acc

        c_own = lax.rem(my + 1, N_DEV)
        ag_bf[0] = out_ref[pl.ds(c_own * SQ, SQ), :].astype(BF)
        a_r = pltpu.make_async_remote_copy(
            src_ref=ag_bf.at[0], dst_ref=ag_bf.at[1],
            send_sem=ag_sems.at[0], recv_sem=ag_sems.at[1],
            device_id=(right,), device_id_type=pl.DeviceIdType.MESH,
        )
        a_l = pltpu.make_async_remote_copy(
            src_ref=ag_bf.at[0], dst_ref=ag_bf.at[2],
            send_sem=ag_sems.at[2], recv_sem=ag_sems.at[3],
            device_id=(left,), device_id_type=pl.DeviceIdType.MESH,
        )
        a_r.start()
        a_l.start()
        a_r.wait()
        fwd = pltpu.make_async_remote_copy(
            src_ref=ag_bf.at[1], dst_ref=ag_bf.at[3],
            send_sem=ag_sems.at[4], recv_sem=ag_sems.at[5],
            device_id=(right,), device_id_type=pl.DeviceIdType.MESH,
        )
        fwd.start()
        out_ref[pl.ds(my * SQ, SQ), :] = ag_bf[1].astype(jnp.float32)
        a_l.wait()
        c2 = lax.rem(my + 2, N_DEV)
        out_ref[pl.ds(c2 * SQ, SQ), :] = ag_bf[2].astype(jnp.float32)
        fwd.wait()
        c3 = lax.rem(my + 3, N_DEV)
        out_ref[pl.ds(c3 * SQ, SQ), :] = ag_bf[3].astype(jnp.float32)

    out = pl.pallas_call(
        body,
        out_shape=jax.ShapeDtypeStruct((B * SQ, D), jnp.float32),
        in_specs=[
            pl.BlockSpec(memory_space=pltpu.VMEM),
            pl.BlockSpec(memory_space=pltpu.VMEM),
            pl.BlockSpec(memory_space=pltpu.VMEM),
            pl.BlockSpec(memory_space=pl.ANY),
            pl.BlockSpec(memory_space=pl.ANY),
        ],
        out_specs=pl.BlockSpec(memory_space=pltpu.VMEM),
        scratch_shapes=[
            pltpu.VMEM((B * SQ, D), BF),
            pltpu.VMEM((D, D), BF),
            pltpu.VMEM((D, D), BF),
            pltpu.VMEM((B * 2, SKV, DH), jnp.float32),
            pltpu.VMEM((B * 2, SKV, DH), jnp.float32),
            pltpu.VMEM((2, SKV, DH), BF),
            pltpu.VMEM((2, SKV, 2 * DH), BF),
            pltpu.VMEM((SQ, D), BF),
            pltpu.VMEM((SQ, D), BF),
            pltpu.VMEM((N_DEV - 1, SQ, D), BF),
            pltpu.VMEM((N_DEV - 1, SQ, D), BF),
            pltpu.VMEM((N_DEV, SQ, D), BF),
            pltpu.SemaphoreType.DMA((B, 4)),
            pltpu.SemaphoreType.DMA((N_DEV - 1,)),
            pltpu.SemaphoreType.DMA((N_DEV - 1,)),
            pltpu.SemaphoreType.DMA((6,)),
        ],
        compiler_params=pltpu.CompilerParams(
            collective_id=0, vmem_limit_bytes=100 * 1024 * 1024),
    )(x2, Wq, Wo, K_ext, V_ext)
    return out.reshape(B, SQ, D)
